# baseline (speedup 1.0000x reference)
import functools

import jax
import jax.numpy as jnp
from jax import lax
from jax.experimental import pallas as pl
from jax.experimental.pallas import tpu as pltpu
from jax.experimental.pallas import tpu_sc as plsc

N = 10000
E = 160000
NPAD = 10240
NC, NS = 2, 16
NW = NC * NS
BE = 128
EPAD = 163840
NB = EPAD // (NW * BE)
RPT = NPAD // NS

MB = 400
D_X, PE, D_IN, H, OUT = 256, 16, 272, 512, 17
OUTP = 32


def _seg_sum(specs, src3, dst3, tc_tiling=True):
    mesh = plsc.VectorSubcoreMesh(
        core_axis_name="c", subcore_axis_name="s",
        num_cores=NC, num_subcores=NS)
    shapes = [(t.shape[0], t.shape[2]) for t, _ in specs]
    out_type = [jax.ShapeDtypeStruct((nc_, NC, NPAD, w), jnp.float32)
                for nc_, w in shapes]
    scratch = [
        pltpu.VMEM((NB, BE), jnp.int32),
        pltpu.VMEM((NB, BE), jnp.int32),
        pltpu.SemaphoreType.DMA,
        pltpu.SemaphoreType.DMA,
        pltpu.SemaphoreType.DMA,
        pltpu.SemaphoreType.DMA,
    ]
    for _, w in shapes:
        scratch.append(pltpu.VMEM((BE, w), jnp.float32))
        scratch.append(pltpu.VMEM((BE, w), jnp.float32))
        scratch.append(pltpu.VMEM_SHARED((NPAD, w), jnp.float32))

    @functools.partial(pl.kernel, out_type=out_type, mesh=mesh,
                       scratch_types=scratch,
                       compiler_params=pltpu.CompilerParams(
                           use_tc_tiling_on_sc=tc_tiling))
    def k(*refs):
        nt = len(specs)
        tabs = refs[:nt]
        src_h, dst_h = refs[nt], refs[nt + 1]
        zs = refs[nt + 2:2 * nt + 2]
        outs = refs[2 * nt + 2:3 * nt + 2]
        src_v, dst_v, sem0, sem1, ssem0, ssem1 = refs[3 * nt + 2:3 * nt + 8]
        bufs = refs[3 * nt + 8:]
        core = lax.axis_index("c")
        sub = lax.axis_index("s")
        wid = sub * NC + core
        pltpu.sync_copy(src_h.at[wid], src_v)
        pltpu.sync_copy(dst_h.at[wid], dst_v)
        for t, (nchunk, _) in enumerate(shapes):
            rows0, rows1 = bufs[3 * t], bufs[3 * t + 1]
            acc = bufs[3 * t + 2]
            z_h, out_t = zs[t], outs[t]
            for c in range(nchunk):
                tab = tabs[t].at[c]
                pltpu.sync_copy(z_h, acc.at[pl.ds(sub * RPT, RPT)])
                plsc.subcore_barrier()
                pltpu.async_copy(tab.at[src_v.at[0]], rows0, sem0)
                pltpu.async_copy(tab.at[src_v.at[1]], rows1, sem1)

                def body(jj, carry):
                    j = jj * 2
                    pltpu.make_async_copy(tab.at[src_v.at[j]], rows0,
                                          sem0).wait()
                    s0 = pltpu.async_copy(rows0, acc.at[dst_v.at[j]], ssem0,
                                          add=True)
                    pltpu.make_async_copy(tab.at[src_v.at[j + 1]], rows1,
                                          sem1).wait()
                    s1 = pltpu.async_copy(rows1, acc.at[dst_v.at[j + 1]],
                                          ssem1, add=True)
                    s0.wait()

                    @pl.when(j + 2 < NB)
                    def _():
                        pltpu.async_copy(tab.at[src_v.at[j + 2]], rows0, sem0)

                    s1.wait()

                    @pl.when(j + 3 < NB)
                    def _():
                        pltpu.async_copy(tab.at[src_v.at[j + 3]], rows1, sem1)
                    return carry

                lax.fori_loop(0, NB // 2, body, 0)
                plsc.subcore_barrier()
                pltpu.sync_copy(acc.at[pl.ds(sub * RPT, RPT)],
                                out_t.at[c].at[core].at[pl.ds(sub * RPT,
                                                              RPT)])

    args = [t for t, _ in specs] + [src3, dst3] + [z for _, z in specs]
    return k(*args)


def _layer0_kernel(a2_ref, a32_ref, x_ref, pe_ref, wl_ref, bl_ref,
                   wr_ref, h1_ref, invd_ref):
    bf = jnp.bfloat16
    deg = a32_ref[0, :, 16:17] + a32_ref[1, :, 16:17]
    invd = 1.0 / jnp.maximum(deg, 1.0)
    agg0 = (a2_ref[0, 0] + a2_ref[0, 1]).astype(bf)
    agg1 = (a2_ref[1, 0] + a2_ref[1, 1]).astype(bf)
    aggpe = (a32_ref[0, :, :16] + a32_ref[1, :, :16]).astype(bf)
    acc = jnp.dot(agg0, wl_ref[0:128], preferred_element_type=jnp.float32)
    acc += jnp.dot(agg1, wl_ref[128:256], preferred_element_type=jnp.float32)
    acc += jnp.dot(aggpe, wl_ref[256:272], preferred_element_type=jnp.float32)
    acc = acc * invd + bl_ref[0]
    acc += jnp.dot(x_ref[...].astype(bf), wr_ref[0:256],
                   preferred_element_type=jnp.float32)
    acc += jnp.dot(pe_ref[...].astype(bf), wr_ref[256:272],
                   preferred_element_type=jnp.float32)
    h = jnp.maximum(acc, 0.0)
    for c in range(4):
        h1_ref[c] = h[:, c * 128:(c + 1) * 128]
    invd_ref[...] = invd


def _layer0(a2, a32, x, pe, wl0t, bl0, wr0t):
    grid = (N // MB,)
    return pl.pallas_call(
        _layer0_kernel,
        grid=grid,
        in_specs=[
            pl.BlockSpec((2, 2, MB, 128), lambda i: (0, 0, i, 0)),
            pl.BlockSpec((2, MB, 32), lambda i: (0, i, 0)),
            pl.BlockSpec((MB, D_X), lambda i: (i, 0)),
            pl.BlockSpec((MB, PE), lambda i: (i, 0)),
            pl.BlockSpec((D_IN, H), lambda i: (0, 0)),
            pl.BlockSpec((1, H), lambda i: (0, 0)),
            pl.BlockSpec((D_IN, H), lambda i: (0, 0)),
        ],
        out_specs=[
            pl.BlockSpec((4, MB, 128), lambda i: (0, i, 0)),
            pl.BlockSpec((MB, 1), lambda i: (i, 0)),
        ],
        out_shape=[
            jax.ShapeDtypeStruct((4, N, 128), jnp.float32),
            jax.ShapeDtypeStruct((N, 1), jnp.float32),
        ],
    )(a2, a32, x, pe, wl0t, bl0, wr0t)


def _layer1_kernel(ah_ref, invd_ref, h1_ref, wl_ref, bl_ref, wr_ref,
                   wl2_ref, wr2_ref, bl2_ref, p2_ref, r2_ref):
    bf = jnp.bfloat16
    invd = invd_ref[...]
    agg = jnp.concatenate([ah_ref[c, 0] + ah_ref[c, 1] for c in range(4)],
                          axis=1) * invd
    h1 = jnp.concatenate([h1_ref[c] for c in range(4)], axis=1)
    h2 = jnp.dot(agg.astype(bf), wl_ref[...], preferred_element_type=jnp.float32)
    h2 += bl_ref[0]
    h2 += jnp.dot(h1.astype(bf), wr_ref[...], preferred_element_type=jnp.float32)
    h2 = jnp.maximum(h2, 0.0).astype(bf)
    p2_ref[...] = jnp.dot(h2, wl2_ref[...], preferred_element_type=jnp.float32)
    r2_ref[...] = jnp.dot(h2, wr2_ref[...],
                          preferred_element_type=jnp.float32) + bl2_ref[0]


def _layer1(ah, invd, h1c, wl1t, bl1, wr1t, wl2pt, wr2pt, bl2p):
    grid = (N // MB,)
    return pl.pallas_call(
        _layer1_kernel,
        grid=grid,
        in_specs=[
            pl.BlockSpec((4, 2, MB, 128), lambda i: (0, 0, i, 0)),
            pl.BlockSpec((MB, 1), lambda i: (i, 0)),
            pl.BlockSpec((4, MB, 128), lambda i: (0, i, 0)),
            pl.BlockSpec((H, H), lambda i: (0, 0)),
            pl.BlockSpec((1, H), lambda i: (0, 0)),
            pl.BlockSpec((H, H), lambda i: (0, 0)),
            pl.BlockSpec((H, OUTP), lambda i: (0, 0)),
            pl.BlockSpec((H, OUTP), lambda i: (0, 0)),
            pl.BlockSpec((1, OUTP), lambda i: (0, 0)),
        ],
        out_specs=[
            pl.BlockSpec((MB, OUTP), lambda i: (i, 0)),
            pl.BlockSpec((MB, OUTP), lambda i: (i, 0)),
        ],
        out_shape=[
            jax.ShapeDtypeStruct((N, OUTP), jnp.float32),
            jax.ShapeDtypeStruct((N, OUTP), jnp.float32),
        ],
    )(ah, invd, h1c, wl1t, bl1, wr1t, wl2pt, wr2pt, bl2p)


def _head_kernel(ap_ref, invd_ref, r2_ref, o1_ref, o2_ref):
    maps = (ap_ref[0] + ap_ref[1]) * invd_ref[...] + r2_ref[...]
    o1_ref[...] = jnp.tanh(maps[:, :16])
    o2_ref[...] = jnp.minimum(jnp.exp(maps[:, 16:17]), 10.0)


def _head(ap, invd, r2):
    grid = (N // MB,)
    return pl.pallas_call(
        _head_kernel,
        grid=grid,
        in_specs=[
            pl.BlockSpec((2, MB, OUTP), lambda i: (0, i, 0)),
            pl.BlockSpec((MB, 1), lambda i: (i, 0)),
            pl.BlockSpec((MB, OUTP), lambda i: (i, 0)),
        ],
        out_specs=[
            pl.BlockSpec((MB, 16), lambda i: (i, 0)),
            pl.BlockSpec((MB, 1), lambda i: (i, 0)),
        ],
        out_shape=[
            jax.ShapeDtypeStruct((N, 16), jnp.float32),
            jax.ShapeDtypeStruct((N, 1), jnp.float32),
        ],
    )(ap, invd, r2)


def kernel(x, pe, edge_index, Wl0, bl0, Wr0, Wl1, bl1, Wr1, Wl2, bl2, Wr2):
    f32 = jnp.float32
    bf = jnp.bfloat16
    src = edge_index[0]
    dst = edge_index[1]
    npe = EPAD - E
    pad_src = jnp.arange(npe, dtype=jnp.int32) % N
    pad_dst = jnp.full((npe,), NPAD - 1, jnp.int32)
    src3 = jnp.concatenate([src, pad_src]).reshape(NW, NB, BE)
    dst3 = jnp.concatenate([dst, pad_dst]).reshape(NW, NB, BE)

    t2 = jnp.stack([x[:, :128], x[:, 128:]])
    t32 = jnp.concatenate([pe, jnp.ones((N, 1), f32),
                           jnp.zeros((N, 15), f32)], axis=1)[None]
    z128 = jnp.zeros((RPT, 128), f32)
    z32 = jnp.zeros((RPT, 32), f32)

    wl0t, wr0t = Wl0.T.astype(bf), Wr0.T.astype(bf)
    wl1t, wr1t = Wl1.T.astype(bf), Wr1.T.astype(bf)
    wl2pt = jnp.zeros((H, OUTP), f32).at[:, :OUT].set(Wl2.T).astype(bf)
    wr2pt = jnp.zeros((H, OUTP), f32).at[:, :OUT].set(Wr2.T).astype(bf)
    bl2p = jnp.zeros((1, OUTP), f32).at[0, :OUT].set(bl2)

    (a2,) = _seg_sum([(t2, z128)], src3, dst3)
    (a32,) = _seg_sum([(t32, z32)], src3, dst3, tc_tiling=False)
    h1c, invd = _layer0(a2, a32[0], x, pe, wl0t, bl0[None], wr0t)

    (ah,) = _seg_sum([(h1c, z128)], src3, dst3)
    p2, r2 = _layer1(ah, invd, h1c, wl1t, bl1[None], wr1t, wl2pt, wr2pt, bl2p)

    (ap,) = _seg_sum([(p2[None], z32)], src3, dst3, tc_tiling=False)
    o1, o2 = _head(ap[0], invd, r2)
    return (o1, o2[:, 0])


# speedup vs baseline: 1.1990x; 1.1990x over previous
import functools

import jax
import jax.numpy as jnp
from jax import lax
from jax.experimental import pallas as pl
from jax.experimental.pallas import tpu as pltpu
from jax.experimental.pallas import tpu_sc as plsc

N = 10000
E = 160000
NPAD = 10240
NC, NS = 2, 16
NW = NC * NS
BE = 128
EPAD = 163840
NB = EPAD // (NW * BE)
RPT = NPAD // NS

MB = 400
D_X, PE, D_IN, H, OUT = 256, 16, 272, 512, 17
OUTP = 32


def _seg_sum(specs, src3, dst3, tc_tiling=True):
    mesh = plsc.VectorSubcoreMesh(
        core_axis_name="c", subcore_axis_name="s",
        num_cores=NC, num_subcores=NS)
    shapes = [(t.shape[0], t.shape[2]) for t, _ in specs]
    out_type = [jax.ShapeDtypeStruct((nc_, NC, NPAD, w), jnp.float32)
                for nc_, w in shapes]
    scratch = [
        pltpu.VMEM((NB, BE), jnp.int32),
        pltpu.VMEM((NB, BE), jnp.int32),
        pltpu.SemaphoreType.DMA,
        pltpu.SemaphoreType.DMA,
    ]
    for _, w in shapes:
        scratch.append(pltpu.VMEM((BE, w), jnp.float32))
        scratch.append(pltpu.VMEM((BE, w), jnp.float32))
        scratch.append(pltpu.VMEM_SHARED((NPAD, w), jnp.float32))

    @functools.partial(pl.kernel, out_type=out_type, mesh=mesh,
                       scratch_types=scratch,
                       compiler_params=pltpu.CompilerParams(
                           use_tc_tiling_on_sc=tc_tiling))
    def k(*refs):
        nt = len(specs)
        tabs = refs[:nt]
        src_h, dst_h = refs[nt], refs[nt + 1]
        zs = refs[nt + 2:2 * nt + 2]
        outs = refs[2 * nt + 2:3 * nt + 2]
        src_v, dst_v, sem0, sem1 = refs[3 * nt + 2:3 * nt + 6]
        bufs = refs[3 * nt + 6:]
        core = lax.axis_index("c")
        sub = lax.axis_index("s")
        wid = sub * NC + core
        pltpu.sync_copy(src_h.at[wid], src_v)
        pltpu.sync_copy(dst_h.at[wid], dst_v)
        for t, (nchunk, _) in enumerate(shapes):
            rows0, rows1 = bufs[3 * t], bufs[3 * t + 1]
            acc = bufs[3 * t + 2]
            z_h, out_t = zs[t], outs[t]
            for c in range(nchunk):
                tab = tabs[t].at[c]
                pltpu.sync_copy(z_h, acc.at[pl.ds(sub * RPT, RPT)])
                plsc.subcore_barrier()
                pltpu.async_copy(tab.at[src_v.at[0]], rows0, sem0)

                def body(jj, carry):
                    j = jj * 2
                    pltpu.async_copy(tab.at[src_v.at[j + 1]], rows1, sem1)
                    pltpu.make_async_copy(tab.at[src_v.at[j]], rows0,
                                          sem0).wait()
                    pltpu.sync_copy(rows0, acc.at[dst_v.at[j]], add=True)

                    @pl.when(j + 2 < NB)
                    def _():
                        pltpu.async_copy(tab.at[src_v.at[j + 2]], rows0, sem0)

                    pltpu.make_async_copy(tab.at[src_v.at[j + 1]], rows1,
                                          sem1).wait()
                    pltpu.sync_copy(rows1, acc.at[dst_v.at[j + 1]], add=True)
                    return carry

                lax.fori_loop(0, NB // 2, body, 0)
                plsc.subcore_barrier()
                pltpu.sync_copy(acc.at[pl.ds(sub * RPT, RPT)],
                                out_t.at[c].at[core].at[pl.ds(sub * RPT,
                                                              RPT)])

    args = [t for t, _ in specs] + [src3, dst3] + [z for _, z in specs]
    return k(*args)


def _layer0_kernel(a2_ref, a32_ref, x_ref, pe_ref, wl_ref, bl_ref,
                   wr_ref, h1_ref, invd_ref):
    bf = jnp.bfloat16
    deg = a32_ref[0, :, 16:17] + a32_ref[1, :, 16:17]
    invd = 1.0 / jnp.maximum(deg, 1.0)
    agg0 = (a2_ref[0, 0] + a2_ref[0, 1]).astype(bf)
    agg1 = (a2_ref[1, 0] + a2_ref[1, 1]).astype(bf)
    aggpe = (a32_ref[0, :, :16] + a32_ref[1, :, :16]).astype(bf)
    acc = jnp.dot(agg0, wl_ref[0:128], preferred_element_type=jnp.float32)
    acc += jnp.dot(agg1, wl_ref[128:256], preferred_element_type=jnp.float32)
    acc += jnp.dot(aggpe, wl_ref[256:272], preferred_element_type=jnp.float32)
    acc = acc * invd + bl_ref[0]
    acc += jnp.dot(x_ref[...].astype(bf), wr_ref[0:256],
                   preferred_element_type=jnp.float32)
    acc += jnp.dot(pe_ref[...].astype(bf), wr_ref[256:272],
                   preferred_element_type=jnp.float32)
    h = jnp.maximum(acc, 0.0)
    for c in range(4):
        h1_ref[c] = h[:, c * 128:(c + 1) * 128]
    invd_ref[...] = invd


def _layer0(a2, a32, x, pe, wl0t, bl0, wr0t):
    grid = (N // MB,)
    return pl.pallas_call(
        _layer0_kernel,
        grid=grid,
        in_specs=[
            pl.BlockSpec((2, 2, MB, 128), lambda i: (0, 0, i, 0)),
            pl.BlockSpec((2, MB, 32), lambda i: (0, i, 0)),
            pl.BlockSpec((MB, D_X), lambda i: (i, 0)),
            pl.BlockSpec((MB, PE), lambda i: (i, 0)),
            pl.BlockSpec((D_IN, H), lambda i: (0, 0)),
            pl.BlockSpec((1, H), lambda i: (0, 0)),
            pl.BlockSpec((D_IN, H), lambda i: (0, 0)),
        ],
        out_specs=[
            pl.BlockSpec((4, MB, 128), lambda i: (0, i, 0)),
            pl.BlockSpec((MB, 1), lambda i: (i, 0)),
        ],
        out_shape=[
            jax.ShapeDtypeStruct((4, N, 128), jnp.float32),
            jax.ShapeDtypeStruct((N, 1), jnp.float32),
        ],
    )(a2, a32, x, pe, wl0t, bl0, wr0t)


def _layer1_kernel(ah_ref, invd_ref, h1_ref, wl_ref, bl_ref, wr_ref,
                   wl2_ref, wr2_ref, bl2_ref, p2_ref, r2_ref):
    bf = jnp.bfloat16
    invd = invd_ref[...]
    agg = jnp.concatenate([ah_ref[c, 0] + ah_ref[c, 1] for c in range(4)],
                          axis=1) * invd
    h1 = jnp.concatenate([h1_ref[c] for c in range(4)], axis=1)
    h2 = jnp.dot(agg.astype(bf), wl_ref[...], preferred_element_type=jnp.float32)
    h2 += bl_ref[0]
    h2 += jnp.dot(h1.astype(bf), wr_ref[...], preferred_element_type=jnp.float32)
    h2 = jnp.maximum(h2, 0.0).astype(bf)
    p2_ref[...] = jnp.dot(h2, wl2_ref[...], preferred_element_type=jnp.float32)
    r2_ref[...] = jnp.dot(h2, wr2_ref[...],
                          preferred_element_type=jnp.float32) + bl2_ref[0]


def _layer1(ah, invd, h1c, wl1t, bl1, wr1t, wl2pt, wr2pt, bl2p):
    grid = (N // MB,)
    return pl.pallas_call(
        _layer1_kernel,
        grid=grid,
        in_specs=[
            pl.BlockSpec((4, 2, MB, 128), lambda i: (0, 0, i, 0)),
            pl.BlockSpec((MB, 1), lambda i: (i, 0)),
            pl.BlockSpec((4, MB, 128), lambda i: (0, i, 0)),
            pl.BlockSpec((H, H), lambda i: (0, 0)),
            pl.BlockSpec((1, H), lambda i: (0, 0)),
            pl.BlockSpec((H, H), lambda i: (0, 0)),
            pl.BlockSpec((H, OUTP), lambda i: (0, 0)),
            pl.BlockSpec((H, OUTP), lambda i: (0, 0)),
            pl.BlockSpec((1, OUTP), lambda i: (0, 0)),
        ],
        out_specs=[
            pl.BlockSpec((MB, OUTP), lambda i: (i, 0)),
            pl.BlockSpec((MB, OUTP), lambda i: (i, 0)),
        ],
        out_shape=[
            jax.ShapeDtypeStruct((N, OUTP), jnp.float32),
            jax.ShapeDtypeStruct((N, OUTP), jnp.float32),
        ],
    )(ah, invd, h1c, wl1t, bl1, wr1t, wl2pt, wr2pt, bl2p)


def _head_kernel(ap_ref, invd_ref, r2_ref, o1_ref, o2_ref):
    maps = (ap_ref[0] + ap_ref[1]) * invd_ref[...] + r2_ref[...]
    o1_ref[...] = jnp.tanh(maps[:, :16])
    o2_ref[...] = jnp.minimum(jnp.exp(maps[:, 16:17]), 10.0)


def _head(ap, invd, r2):
    grid = (N // MB,)
    return pl.pallas_call(
        _head_kernel,
        grid=grid,
        in_specs=[
            pl.BlockSpec((2, MB, OUTP), lambda i: (0, i, 0)),
            pl.BlockSpec((MB, 1), lambda i: (i, 0)),
            pl.BlockSpec((MB, OUTP), lambda i: (i, 0)),
        ],
        out_specs=[
            pl.BlockSpec((MB, 16), lambda i: (i, 0)),
            pl.BlockSpec((MB, 1), lambda i: (i, 0)),
        ],
        out_shape=[
            jax.ShapeDtypeStruct((N, 16), jnp.float32),
            jax.ShapeDtypeStruct((N, 1), jnp.float32),
        ],
    )(ap, invd, r2)


def kernel(x, pe, edge_index, Wl0, bl0, Wr0, Wl1, bl1, Wr1, Wl2, bl2, Wr2):
    f32 = jnp.float32
    bf = jnp.bfloat16
    src = edge_index[0]
    dst = edge_index[1]
    npe = EPAD - E
    pad_src = jnp.arange(npe, dtype=jnp.int32) % N
    pad_dst = jnp.full((npe,), NPAD - 1, jnp.int32)
    src3 = jnp.concatenate([src, pad_src]).reshape(NW, NB, BE)
    dst3 = jnp.concatenate([dst, pad_dst]).reshape(NW, NB, BE)

    t2 = jnp.stack([x[:, :128], x[:, 128:]])
    t32 = jnp.concatenate([pe, jnp.ones((N, 1), f32),
                           jnp.zeros((N, 15), f32)], axis=1)[None]
    z128 = jnp.zeros((RPT, 128), f32)
    z32 = jnp.zeros((RPT, 32), f32)

    wl0t, wr0t = Wl0.T.astype(bf), Wr0.T.astype(bf)
    wl1t, wr1t = Wl1.T.astype(bf), Wr1.T.astype(bf)
    wl2pt = jnp.zeros((H, OUTP), f32).at[:, :OUT].set(Wl2.T).astype(bf)
    wr2pt = jnp.zeros((H, OUTP), f32).at[:, :OUT].set(Wr2.T).astype(bf)
    bl2p = jnp.zeros((1, OUTP), f32).at[0, :OUT].set(bl2)

    (a2,) = _seg_sum([(t2, z128)], src3, dst3)
    (a32,) = _seg_sum([(t32, z32)], src3, dst3, tc_tiling=False)
    h1c, invd = _layer0(a2, a32[0], x, pe, wl0t, bl0[None], wr0t)

    (ah,) = _seg_sum([(h1c, z128)], src3, dst3)
    p2, r2 = _layer1(ah, invd, h1c, wl1t, bl1[None], wr1t, wl2pt, wr2pt, bl2p)

    (ap,) = _seg_sum([(p2[None], z32)], src3, dst3, tc_tiling=False)
    o1, o2 = _head(ap[0], invd, r2)
    return (o1, o2[:, 0])
